# Initial kernel scaffold; baseline (speedup 1.0000x reference)
import jax
import jax.numpy as jnp
from jax.experimental import pallas as pl


def kernel(features, edge_index, W_emb, b_emb, W_self, W_neigh, b_sage, bn_gamma, bn_beta, bn_mean, bn_var, W1, b1, W2, b2):
    raise NotImplementedError("write your pallas kernel here")


# speedup vs baseline: 6.0454x; 6.0454x over previous
import functools
import jax
import jax.numpy as jnp
from jax import lax
from jax.experimental import pallas as pl
from jax.experimental.pallas import tpu as pltpu
from jax.experimental.pallas import tpu_sc as plsc

N = 50000
E = 800000
IN_DIM = 12
HID = 64
HALF = 32
ODIM = 2
NLAYERS = 3

NC = 2
NS = 16
EPT = E // NS
CH = 400
NCHUNK = EPT // CH
RPT = 3128
LAST_BASE = N - RPT
NZ = RPT // CH
REM = RPT - NZ * CH

@functools.cache
def _sc_kernels():
    mesh = plsc.VectorSubcoreMesh(core_axis_name="c", subcore_axis_name="s",
                                  num_cores=NC, num_subcores=NS)
    agg = functools.partial(
        pl.kernel,
        out_type=jax.ShapeDtypeStruct((NC, N, HALF), jnp.float32),
        mesh=mesh,
        compiler_params=pltpu.CompilerParams(use_tc_tiling_on_sc=False),
        scratch_types=[
            pltpu.VMEM((CH,), jnp.int32),
            pltpu.VMEM((CH,), jnp.int32),
            pltpu.VMEM((CH, HALF), jnp.float32),
            pltpu.VMEM_SHARED((N, HALF), jnp.float32),
            pltpu.SemaphoreType.DMA,
        ],
    )(_agg_body)
    deg = functools.partial(
        pl.kernel,
        out_type=jax.ShapeDtypeStruct((N,), jnp.float32),
        mesh=mesh,
        scratch_types=[
            pltpu.VMEM((CH,), jnp.int32),
            pltpu.VMEM((CH,), jnp.float32),
            pltpu.VMEM((CH,), jnp.float32),
            pltpu.VMEM((RPT,), jnp.float32),
            pltpu.VMEM_SHARED((N,), jnp.float32),
        ],
    )(_deg_body)
    return agg, deg


def _agg_body(table, srcoff, dsti, out, src_v, dst_v, rows_v, acc, sem):
    c = lax.axis_index("c")
    s = lax.axis_index("s")
    zero16 = jnp.zeros((16,), jnp.float32)

    def zrow(j, carry):
        rows_v[j, pl.ds(0, 16)] = zero16
        rows_v[j, pl.ds(16, 16)] = zero16
        return carry

    lax.fori_loop(0, CH, zrow, 0)

    base = pl.multiple_of(jnp.where(s == NS - 1, LAST_BASE, s * RPT), 8)

    def zcp(j, carry):
        pltpu.sync_copy(rows_v, acc.at[pl.ds(base + j * CH, CH)])
        return carry

    lax.fori_loop(0, NZ, zcp, 0)
    pltpu.sync_copy(rows_v.at[pl.ds(0, REM)], acc.at[pl.ds(base + NZ * CH, REM)])
    plsc.subcore_barrier()

    def chunk(k, carry):
        off = pl.multiple_of(s * EPT + k * CH, 8)
        pltpu.sync_copy(srcoff.at[pl.ds(c * E + off, CH)], src_v)
        pltpu.sync_copy(dsti.at[pl.ds(off, CH)], dst_v)
        pltpu.async_copy(table.at[src_v], rows_v, sem).wait()
        pltpu.sync_copy(rows_v, acc.at[dst_v], add=True)
        return carry

    lax.fori_loop(0, NCHUNK, chunk, 0)

    plsc.subcore_barrier()
    pltpu.sync_copy(acc.at[pl.ds(base, RPT)], out.at[c, pl.ds(base, RPT)])


def _deg_body(dsti, out, dst_v, ones_v, zeros_v, row_v, acc):
    s = lax.axis_index("s")
    one16 = jnp.ones((16,), jnp.float32)
    zero16 = jnp.zeros((16,), jnp.float32)

    def fill(j, carry):
        ones_v[pl.ds(j * 16, 16)] = one16
        zeros_v[pl.ds(j * 16, 16)] = zero16
        return carry

    lax.fori_loop(0, CH // 16, fill, 0)

    base = pl.multiple_of(jnp.where(s == NS - 1, LAST_BASE, s * RPT), 8)

    def zcp(j, carry):
        pltpu.sync_copy(zeros_v, acc.at[pl.ds(base + j * CH, CH)])
        return carry

    lax.fori_loop(0, NZ, zcp, 0)
    pltpu.sync_copy(zeros_v.at[pl.ds(0, REM)], acc.at[pl.ds(base + NZ * CH, REM)])
    plsc.subcore_barrier()

    def chunk(k, carry):
        off = pl.multiple_of(s * EPT + k * CH, 8)
        pltpu.sync_copy(dsti.at[pl.ds(off, CH)], dst_v)
        pltpu.sync_copy(ones_v, acc.at[dst_v], add=True)
        return carry

    lax.fori_loop(0, NCHUNK, chunk, 0)

    plsc.subcore_barrier()
    pltpu.sync_copy(acc.at[pl.ds(base, RPT)], row_v)
    pltpu.sync_copy(row_v, out.at[pl.ds(base, RPT)])


BLK = 1000
GRID = N // BLK


def _embed_body(f_ref, w_ref, b_ref, out_ref):
    x = jnp.dot(f_ref[...], w_ref[...], preferred_element_type=jnp.float32) + b_ref[...]
    out_ref[0] = x[:, :HALF]
    out_ref[1] = x[:, HALF:]


def _layer_body(h_ref, hn_ref, deg_ref, wsA, wsB, wnA, wnB, b_ref, out_ref, *, residual):
    tl = h_ref[0]
    th = h_ref[1]
    invd = 1.0 / jnp.maximum(deg_ref[...], 1.0)
    x = (jnp.dot(tl, wsA[...], preferred_element_type=jnp.float32)
         + jnp.dot(th, wsB[...], preferred_element_type=jnp.float32)
         + (jnp.dot(hn_ref[0], wnA[...], preferred_element_type=jnp.float32)
            + jnp.dot(hn_ref[1], wnB[...], preferred_element_type=jnp.float32)) * invd
         + b_ref[...])
    x = jnp.maximum(x, 0.0)
    if residual:
        out_ref[0] = x[:, :HALF] + tl
        out_ref[1] = x[:, HALF:] + th
    else:
        out_ref[0] = x[:, :HALF]
        out_ref[1] = x[:, HALF:]


def _head_body(h_ref, w1A, w1B, b1_ref, w2_ref, b2_ref, out_ref):
    x = (jnp.dot(h_ref[0], w1A[...], preferred_element_type=jnp.float32)
         + jnp.dot(h_ref[1], w1B[...], preferred_element_type=jnp.float32)
         + b1_ref[...])
    x = jnp.maximum(x, 0.0)
    out_ref[...] = jnp.dot(x, w2_ref[...], preferred_element_type=jnp.float32) + b2_ref[...]


def _full(shape):
    return pl.BlockSpec(shape, lambda i: tuple(0 for _ in shape))


def _embed_call(features, weT, be):
    return pl.pallas_call(
        _embed_body,
        grid=(GRID,),
        in_specs=[pl.BlockSpec((BLK, IN_DIM), lambda i: (i, 0)),
                  _full((IN_DIM, HID)), _full((1, HID))],
        out_specs=pl.BlockSpec((2, BLK, HALF), lambda i: (0, i, 0)),
        out_shape=jax.ShapeDtypeStruct((2, N, HALF), jnp.float32),
    )(features, weT, be)


def _layer_call(residual, h, hn, deg, wsA, wsB, wnA, wnB, bf):
    return pl.pallas_call(
        functools.partial(_layer_body, residual=residual),
        grid=(GRID,),
        in_specs=[pl.BlockSpec((2, BLK, HALF), lambda i: (0, i, 0)),
                  pl.BlockSpec((2, BLK, HALF), lambda i: (0, i, 0)),
                  pl.BlockSpec((BLK, 1), lambda i: (i, 0)),
                  _full((HALF, HID)), _full((HALF, HID)),
                  _full((HALF, HID)), _full((HALF, HID)), _full((1, HID))],
        out_specs=pl.BlockSpec((2, BLK, HALF), lambda i: (0, i, 0)),
        out_shape=jax.ShapeDtypeStruct((2, N, HALF), jnp.float32),
    )(h, hn, deg, wsA, wsB, wnA, wnB, bf)


def _head_call(h, w1A, w1B, b1r, w2T, b2r):
    return pl.pallas_call(
        _head_body,
        grid=(GRID,),
        in_specs=[pl.BlockSpec((2, BLK, HALF), lambda i: (0, i, 0)),
                  _full((HALF, HALF)), _full((HALF, HALF)), _full((1, HALF)),
                  _full((HALF, ODIM)), _full((1, ODIM))],
        out_specs=pl.BlockSpec((BLK, ODIM), lambda i: (i, 0)),
        out_shape=jax.ShapeDtypeStruct((N, ODIM), jnp.float32),
    )(h, w1A, w1B, b1r, w2T, b2r)


def kernel(features, edge_index, W_emb, b_emb, W_self, W_neigh, b_sage,
           bn_gamma, bn_beta, bn_mean, bn_var, W1, b1, W2, b2):
    ei = edge_index.astype(jnp.int32)
    src = ei[0]
    dst = ei[1]
    srcoff = jnp.concatenate([src, src + N])

    scale = bn_gamma * lax.rsqrt(bn_var + 1e-5)
    bf = (b_sage - bn_mean) * scale + bn_beta
    Wsf = W_self * scale[:, :, None]
    Wnf = W_neigh * scale[:, :, None]
    wsA = jnp.transpose(Wsf[:, :, :HALF], (0, 2, 1))
    wsB = jnp.transpose(Wsf[:, :, HALF:], (0, 2, 1))
    wnA = jnp.transpose(Wnf[:, :, :HALF], (0, 2, 1))
    wnB = jnp.transpose(Wnf[:, :, HALF:], (0, 2, 1))

    agg, degk = _sc_kernels()
    h = _embed_call(features, W_emb.T, b_emb[None, :])
    deg = degk(dst).reshape(N, 1)
    for i in range(NLAYERS):
        hn = agg(h.reshape(NC * N, HALF), srcoff, dst)
        h = _layer_call(i > 0, h, hn, deg,
                        wsA[i], wsB[i], wnA[i], wnB[i], bf[i][None, :])
    return _head_call(h, W1[:, :HALF].T, W1[:, HALF:].T, b1[None, :],
                      W2.T, b2[None, :])


# speedup vs baseline: 8.7245x; 1.4432x over previous
import functools
import jax
import jax.numpy as jnp
from jax import lax
from jax.experimental import pallas as pl
from jax.experimental.pallas import tpu as pltpu
from jax.experimental.pallas import tpu_sc as plsc

N = 50000
E = 800000
IN_DIM = 12
HID = 64
HALF = 32
ODIM = 2
NLAYERS = 3

NC = 2
NS = 16
EPT = E // NS
CH = 400
NCHUNK = EPT // CH
RPT = 3128
LAST_BASE = N - RPT
NZ = RPT // CH
REM = RPT - NZ * CH

@functools.cache
def _sc_kernels():
    mesh = plsc.VectorSubcoreMesh(core_axis_name="c", subcore_axis_name="s",
                                  num_cores=NC, num_subcores=NS)
    agg = functools.partial(
        pl.kernel,
        out_type=jax.ShapeDtypeStruct((NC, N, HALF), jnp.float32),
        mesh=mesh,
        compiler_params=pltpu.CompilerParams(use_tc_tiling_on_sc=False),
        scratch_types=[
            pltpu.VMEM((CH,), jnp.int32),
            pltpu.VMEM((CH,), jnp.int32),
            pltpu.VMEM((CH, HALF), jnp.float32),
            pltpu.VMEM((CH,), jnp.int32),
            pltpu.VMEM((CH,), jnp.int32),
            pltpu.VMEM((CH, HALF), jnp.float32),
            pltpu.VMEM_SHARED((N, HALF), jnp.float32),
            pltpu.SemaphoreType.DMA,
            pltpu.SemaphoreType.DMA,
            pltpu.SemaphoreType.DMA,
            pltpu.SemaphoreType.DMA,
            pltpu.SemaphoreType.DMA,
            pltpu.SemaphoreType.DMA,
        ],
    )(_agg_body)
    deg = functools.partial(
        pl.kernel,
        out_type=jax.ShapeDtypeStruct((N,), jnp.float32),
        mesh=mesh,
        scratch_types=[
            pltpu.VMEM((CH,), jnp.int32),
            pltpu.VMEM((CH,), jnp.float32),
            pltpu.VMEM((CH,), jnp.float32),
            pltpu.VMEM((RPT,), jnp.float32),
            pltpu.VMEM_SHARED((N,), jnp.float32),
        ],
    )(_deg_body)
    return agg, deg


def _agg_body(table, srcoff, dsti, out,
              src0, dst0, rows0, src1, dst1, rows1, acc,
              semI0, semG0, semS0, semI1, semG1, semS1):
    c = lax.axis_index("c")
    s = lax.axis_index("s")
    zero16 = jnp.zeros((16,), jnp.float32)

    def zrow(j, carry):
        rows0[j, pl.ds(0, 16)] = zero16
        rows0[j, pl.ds(16, 16)] = zero16
        return carry

    lax.fori_loop(0, CH, zrow, 0)

    base = pl.multiple_of(jnp.where(s == NS - 1, LAST_BASE, s * RPT), 8)

    def zcp(j, carry):
        pltpu.sync_copy(rows0, acc.at[pl.ds(base + j * CH, CH)])
        return carry

    lax.fori_loop(0, NZ, zcp, 0)
    pltpu.sync_copy(rows0.at[pl.ds(0, REM)], acc.at[pl.ds(base + NZ * CH, REM)])
    plsc.subcore_barrier()

    ebase = s * EPT
    slots = ((src0, dst0, rows0, semI0, semG0, semS0),
             (src1, dst1, rows1, semI1, semG1, semS1))

    def issue_i(k, sl):
        off = pl.multiple_of(ebase + k * CH, 8)
        pltpu.async_copy(srcoff.at[pl.ds(c * E + off, CH)], sl[0], sl[3])
        pltpu.async_copy(dsti.at[pl.ds(off, CH)], sl[1], sl[3])

    def wait_i(sl):
        pltpu.make_async_copy(srcoff.at[pl.ds(0, CH)], sl[0], sl[3]).wait()
        pltpu.make_async_copy(dsti.at[pl.ds(0, CH)], sl[1], sl[3]).wait()

    def issue_g(sl):
        pltpu.async_copy(table.at[sl[0]], sl[2], sl[4])

    def wait_g(sl):
        pltpu.make_async_copy(table.at[sl[0]], sl[2], sl[4]).wait()

    def issue_s(sl):
        pltpu.async_copy(sl[2], acc.at[sl[1]], sl[5], add=True)

    def wait_s(sl):
        pltpu.make_async_copy(sl[2], acc.at[sl[1]], sl[5]).wait()

    issue_i(0, slots[0])
    wait_i(slots[0])
    issue_g(slots[0])
    issue_i(1, slots[1])
    wait_g(slots[0])
    issue_s(slots[0])

    def pair(p, carry):
        for b in (1, 0):
            k = 2 * p + (1 if b == 1 else 2)
            sl = slots[b]
            ot = slots[1 - b]
            wait_i(sl)
            issue_g(sl)
            wait_s(ot)
            issue_i(k + 1, ot)
            wait_g(sl)
            issue_s(sl)
        return carry

    lax.fori_loop(0, (NCHUNK - 1) // 2, pair, 0)

    wait_s(slots[0])
    wait_i(slots[1])
    plsc.subcore_barrier()
    pltpu.sync_copy(acc.at[pl.ds(base, RPT)], out.at[c, pl.ds(base, RPT)])


def _deg_body(dsti, out, dst_v, ones_v, zeros_v, row_v, acc):
    s = lax.axis_index("s")
    one16 = jnp.ones((16,), jnp.float32)
    zero16 = jnp.zeros((16,), jnp.float32)

    def fill(j, carry):
        ones_v[pl.ds(j * 16, 16)] = one16
        zeros_v[pl.ds(j * 16, 16)] = zero16
        return carry

    lax.fori_loop(0, CH // 16, fill, 0)

    base = pl.multiple_of(jnp.where(s == NS - 1, LAST_BASE, s * RPT), 8)

    def zcp(j, carry):
        pltpu.sync_copy(zeros_v, acc.at[pl.ds(base + j * CH, CH)])
        return carry

    lax.fori_loop(0, NZ, zcp, 0)
    pltpu.sync_copy(zeros_v.at[pl.ds(0, REM)], acc.at[pl.ds(base + NZ * CH, REM)])
    plsc.subcore_barrier()

    def chunk(k, carry):
        off = pl.multiple_of(s * EPT + k * CH, 8)
        pltpu.sync_copy(dsti.at[pl.ds(off, CH)], dst_v)
        pltpu.sync_copy(ones_v, acc.at[dst_v], add=True)
        return carry

    lax.fori_loop(0, NCHUNK, chunk, 0)

    plsc.subcore_barrier()
    pltpu.sync_copy(acc.at[pl.ds(base, RPT)], row_v)
    pltpu.sync_copy(row_v, out.at[pl.ds(base, RPT)])


BLK = 1000
GRID = N // BLK


def _embed_body(f_ref, w_ref, b_ref, out_ref):
    x = jnp.dot(f_ref[...], w_ref[...], preferred_element_type=jnp.float32) + b_ref[...]
    out_ref[0] = x[:, :HALF]
    out_ref[1] = x[:, HALF:]


def _layer_body(h_ref, hn_ref, deg_ref, wsA, wsB, wnA, wnB, b_ref, out_ref, *, residual):
    tl = h_ref[0]
    th = h_ref[1]
    invd = 1.0 / jnp.maximum(deg_ref[...], 1.0)
    x = (jnp.dot(tl, wsA[...], preferred_element_type=jnp.float32)
         + jnp.dot(th, wsB[...], preferred_element_type=jnp.float32)
         + (jnp.dot(hn_ref[0], wnA[...], preferred_element_type=jnp.float32)
            + jnp.dot(hn_ref[1], wnB[...], preferred_element_type=jnp.float32)) * invd
         + b_ref[...])
    x = jnp.maximum(x, 0.0)
    if residual:
        out_ref[0] = x[:, :HALF] + tl
        out_ref[1] = x[:, HALF:] + th
    else:
        out_ref[0] = x[:, :HALF]
        out_ref[1] = x[:, HALF:]


def _head_body(h_ref, w1A, w1B, b1_ref, w2_ref, b2_ref, out_ref):
    x = (jnp.dot(h_ref[0], w1A[...], preferred_element_type=jnp.float32)
         + jnp.dot(h_ref[1], w1B[...], preferred_element_type=jnp.float32)
         + b1_ref[...])
    x = jnp.maximum(x, 0.0)
    out_ref[...] = jnp.dot(x, w2_ref[...], preferred_element_type=jnp.float32) + b2_ref[...]


def _full(shape):
    return pl.BlockSpec(shape, lambda i: tuple(0 for _ in shape))


def _embed_call(features, weT, be):
    return pl.pallas_call(
        _embed_body,
        grid=(GRID,),
        in_specs=[pl.BlockSpec((BLK, IN_DIM), lambda i: (i, 0)),
                  _full((IN_DIM, HID)), _full((1, HID))],
        out_specs=pl.BlockSpec((2, BLK, HALF), lambda i: (0, i, 0)),
        out_shape=jax.ShapeDtypeStruct((2, N, HALF), jnp.float32),
    )(features, weT, be)


def _layer_call(residual, h, hn, deg, wsA, wsB, wnA, wnB, bf):
    return pl.pallas_call(
        functools.partial(_layer_body, residual=residual),
        grid=(GRID,),
        in_specs=[pl.BlockSpec((2, BLK, HALF), lambda i: (0, i, 0)),
                  pl.BlockSpec((2, BLK, HALF), lambda i: (0, i, 0)),
                  pl.BlockSpec((BLK, 1), lambda i: (i, 0)),
                  _full((HALF, HID)), _full((HALF, HID)),
                  _full((HALF, HID)), _full((HALF, HID)), _full((1, HID))],
        out_specs=pl.BlockSpec((2, BLK, HALF), lambda i: (0, i, 0)),
        out_shape=jax.ShapeDtypeStruct((2, N, HALF), jnp.float32),
    )(h, hn, deg, wsA, wsB, wnA, wnB, bf)


def _head_call(h, w1A, w1B, b1r, w2T, b2r):
    return pl.pallas_call(
        _head_body,
        grid=(GRID,),
        in_specs=[pl.BlockSpec((2, BLK, HALF), lambda i: (0, i, 0)),
                  _full((HALF, HALF)), _full((HALF, HALF)), _full((1, HALF)),
                  _full((HALF, ODIM)), _full((1, ODIM))],
        out_specs=pl.BlockSpec((BLK, ODIM), lambda i: (i, 0)),
        out_shape=jax.ShapeDtypeStruct((N, ODIM), jnp.float32),
    )(h, w1A, w1B, b1r, w2T, b2r)


def kernel(features, edge_index, W_emb, b_emb, W_self, W_neigh, b_sage,
           bn_gamma, bn_beta, bn_mean, bn_var, W1, b1, W2, b2):
    ei = edge_index.astype(jnp.int32)
    src = ei[0]
    dst = ei[1]
    srcoff = jnp.concatenate([src, src + N, jnp.zeros((CH,), jnp.int32)])
    dst = jnp.concatenate([dst, jnp.zeros((CH,), jnp.int32)])

    scale = bn_gamma * lax.rsqrt(bn_var + 1e-5)
    bf = (b_sage - bn_mean) * scale + bn_beta
    Wsf = W_self * scale[:, :, None]
    Wnf = W_neigh * scale[:, :, None]
    wsA = jnp.transpose(Wsf[:, :, :HALF], (0, 2, 1))
    wsB = jnp.transpose(Wsf[:, :, HALF:], (0, 2, 1))
    wnA = jnp.transpose(Wnf[:, :, :HALF], (0, 2, 1))
    wnB = jnp.transpose(Wnf[:, :, HALF:], (0, 2, 1))

    agg, degk = _sc_kernels()
    h = _embed_call(features, W_emb.T, b_emb[None, :])
    deg = degk(dst).reshape(N, 1)
    for i in range(NLAYERS):
        hn = agg(h.reshape(NC * N, HALF), srcoff, dst)
        h = _layer_call(i > 0, h, hn, deg,
                        wsA[i], wsB[i], wnA[i], wnB[i], bf[i][None, :])
    return _head_call(h, W1[:, :HALF].T, W1[:, HALF:].T, b1[None, :],
                      W2.T, b2[None, :])


# speedup vs baseline: 12.6040x; 1.4447x over previous
import functools
import jax
import jax.numpy as jnp
from jax import lax
from jax.experimental import pallas as pl
from jax.experimental.pallas import tpu as pltpu
from jax.experimental.pallas import tpu_sc as plsc

N = 50000
E = 800000
IN_DIM = 12
HID = 64
HALF = 32
ODIM = 2
NLAYERS = 3

NP = 51200
NPQ = NP // 4

NC = 2
NS = 16
EPT = E // NS
CH = 400
NCHUNK = EPT // CH
RPT = 3128
LAST_BASE = N - RPT
NZ = RPT // CH
REM = RPT - NZ * CH
RPTP = NP // NS
NPAD_CH = (NP - N) // CH


def _agg_body(tab_lo, tab_hi, srcp, dstp, *refs, with_deg):
    if with_deg:
        (out_lo, out_hi, deg_out,
         src0, dst0, rows0, src1, dst1, rows1, ones_v, acc, acc_deg,
         semI0, semG0, semS0, semI1, semG1, semS1) = refs
    else:
        (out_lo, out_hi,
         src0, dst0, rows0, src1, dst1, rows1, acc,
         semI0, semG0, semS0, semI1, semG1, semS1) = refs
    c = lax.axis_index("c")
    s = lax.axis_index("s")
    zero16 = jnp.zeros((16,), jnp.float32)
    one16 = jnp.ones((16,), jnp.float32)

    def zrow(j, carry):
        rows0[j, pl.ds(0, 16)] = zero16
        rows0[j, pl.ds(16, 16)] = zero16
        return carry

    lax.fori_loop(0, CH, zrow, 0)

    base = pl.multiple_of(jnp.where(s == NS - 1, LAST_BASE, s * RPT), 8)

    def zcp(j, carry):
        pltpu.sync_copy(rows0, acc.at[pl.ds(base + j * CH, CH)])
        return carry

    lax.fori_loop(0, NZ, zcp, 0)
    pltpu.sync_copy(rows0.at[pl.ds(0, REM)], acc.at[pl.ds(base + NZ * CH, REM)])

    if with_deg:
        def fill0(j, carry):
            ones_v[pl.ds(j * 16, 16)] = zero16
            return carry

        lax.fori_loop(0, CH // 16, fill0, 0)
        basep = pl.multiple_of(s * RPTP, 8)

        def zdeg(j, carry):
            pltpu.sync_copy(ones_v, acc_deg.at[pl.ds(basep + j * CH, CH)])
            return carry

        lax.fori_loop(0, RPTP // CH, zdeg, 0)

        def fill1(j, carry):
            ones_v[pl.ds(j * 16, 16)] = one16
            return carry

        lax.fori_loop(0, CH // 16, fill1, 0)

    plsc.subcore_barrier()

    ebase = s * EPT
    slots = ((src0, dst0, rows0, semI0, semG0, semS0),
             (src1, dst1, rows1, semI1, semG1, semS1))

    def issue_i(k, sl):
        off = pl.multiple_of(ebase + k * CH, 8)
        pltpu.async_copy(srcp.at[pl.ds(off, CH)], sl[0], sl[3])
        pltpu.async_copy(dstp.at[pl.ds(off, CH)], sl[1], sl[3])

    def wait_i(sl):
        pltpu.make_async_copy(srcp.at[pl.ds(0, CH)], sl[0], sl[3]).wait()
        pltpu.make_async_copy(dstp.at[pl.ds(0, CH)], sl[1], sl[3]).wait()

    def issue_g(sl):
        @pl.when(c == 0)
        def _():
            pltpu.async_copy(tab_lo.at[sl[0]], sl[2], sl[4])

        @pl.when(c == 1)
        def _():
            pltpu.async_copy(tab_hi.at[sl[0]], sl[2], sl[4])

    def wait_g(sl):
        pltpu.make_async_copy(tab_lo.at[sl[0]], sl[2], sl[4]).wait()

    def issue_s(sl):
        pltpu.async_copy(sl[2], acc.at[sl[1]], sl[5], add=True)
        if with_deg:
            pltpu.async_copy(ones_v, acc_deg.at[sl[1]], sl[5], add=True)

    def wait_s(sl):
        pltpu.make_async_copy(sl[2], acc.at[sl[1]], sl[5]).wait()
        if with_deg:
            pltpu.make_async_copy(ones_v, acc_deg.at[sl[1]], sl[5]).wait()

    issue_i(0, slots[0])
    wait_i(slots[0])
    issue_g(slots[0])
    issue_i(1, slots[1])
    wait_g(slots[0])
    issue_s(slots[0])

    def pair(p, carry):
        for b in (1, 0):
            k = 2 * p + (1 if b == 1 else 2)
            sl = slots[b]
            ot = slots[1 - b]
            wait_i(sl)
            issue_g(sl)
            wait_s(ot)
            issue_i(k + 1, ot)
            wait_g(sl)
            issue_s(sl)
        return carry

    lax.fori_loop(0, (NCHUNK - 1) // 2, pair, 0)

    wait_s(slots[0])
    wait_i(slots[1])
    plsc.subcore_barrier()

    @pl.when(c == 0)
    def _():
        pltpu.sync_copy(acc.at[pl.ds(base, RPT)], out_lo.at[pl.ds(base, RPT)])

    @pl.when(c == 1)
    def _():
        pltpu.sync_copy(acc.at[pl.ds(base, RPT)], out_hi.at[pl.ds(base, RPT)])

    if with_deg:
        basep = pl.multiple_of(s * RPTP, 8)

        def wdeg(j, carry):
            pltpu.sync_copy(acc_deg.at[pl.ds(basep + j * CH, CH)], ones_v)
            pltpu.sync_copy(ones_v, deg_out.at[pl.ds(basep + j * CH, CH)])
            return carry

        lax.fori_loop(0, RPTP // CH, wdeg, 0)

    @pl.when(s == 0)
    def _():
        lax.fori_loop(0, CH, zrow, 0)

        def pz(j, carry):
            @pl.when(c == 0)
            def _():
                pltpu.sync_copy(rows0, out_lo.at[pl.ds(N + j * CH, CH)])

            @pl.when(c == 1)
            def _():
                pltpu.sync_copy(rows0, out_hi.at[pl.ds(N + j * CH, CH)])

            return carry

        lax.fori_loop(0, NPAD_CH, pz, 0)


@functools.cache
def _sc_kernels():
    mesh = plsc.VectorSubcoreMesh(core_axis_name="c", subcore_axis_name="s",
                                  num_cores=NC, num_subcores=NS)
    tab = jax.ShapeDtypeStruct((NP, HALF), jnp.float32)
    sems = [pltpu.SemaphoreType.DMA] * 6
    slot_bufs = [
        pltpu.VMEM((CH,), jnp.int32),
        pltpu.VMEM((CH,), jnp.int32),
        pltpu.VMEM((CH, HALF), jnp.float32),
        pltpu.VMEM((CH,), jnp.int32),
        pltpu.VMEM((CH,), jnp.int32),
        pltpu.VMEM((CH, HALF), jnp.float32),
    ]
    agg0 = functools.partial(
        pl.kernel,
        out_type=[tab, tab, jax.ShapeDtypeStruct((NP,), jnp.float32)],
        mesh=mesh,
        compiler_params=pltpu.CompilerParams(use_tc_tiling_on_sc=False),
        scratch_types=slot_bufs + [
            pltpu.VMEM((CH,), jnp.float32),
            pltpu.VMEM_SHARED((N, HALF), jnp.float32),
            pltpu.VMEM_SHARED((NP,), jnp.float32),
        ] + sems,
    )(functools.partial(_agg_body, with_deg=True))
    agg = functools.partial(
        pl.kernel,
        out_type=[tab, tab],
        mesh=mesh,
        compiler_params=pltpu.CompilerParams(use_tc_tiling_on_sc=False),
        scratch_types=slot_bufs + [
            pltpu.VMEM_SHARED((N, HALF), jnp.float32),
        ] + sems,
    )(functools.partial(_agg_body, with_deg=False))
    return agg0, agg


BROW = 320
GRID = NPQ // BROW


def _embed_body(f_ref, p_ref, q_ref, blo_ref, bhi_ref, lo_ref, hi_ref):
    f = f_ref[...]
    lo_ref[...] = jnp.dot(f, p_ref[...], preferred_element_type=jnp.float32) + blo_ref[...]
    hi_ref[...] = jnp.dot(f, q_ref[...], preferred_element_type=jnp.float32) + bhi_ref[...]


def _layer_body(tl_ref, th_ref, nl_ref, nh_ref, dg_ref,
                sa, sb, sc_, sd, na, nb, ncc, nd, blo_ref, bhi_ref,
                lo_ref, hi_ref, *, residual):
    tl = tl_ref[...]
    th = th_ref[...]
    nl = nl_ref[...]
    nh = nh_ref[...]
    invd = 1.0 / jnp.maximum(dg_ref[...], 1.0)
    dot = functools.partial(jnp.dot, preferred_element_type=jnp.float32)
    xlo = dot(tl, sa[...]) + dot(th, sb[...]) + (dot(nl, na[...]) + dot(nh, nb[...])) * invd + blo_ref[...]
    xhi = dot(tl, sc_[...]) + dot(th, sd[...]) + (dot(nl, ncc[...]) + dot(nh, nd[...])) * invd + bhi_ref[...]
    xlo = jnp.maximum(xlo, 0.0)
    xhi = jnp.maximum(xhi, 0.0)
    if residual:
        xlo = xlo + tl
        xhi = xhi + th
    lo_ref[...] = xlo
    hi_ref[...] = xhi


def _head_body(tl_ref, th_ref, w1a, w1b, b1_ref, w2_ref, b2_ref, out_ref):
    dot = functools.partial(jnp.dot, preferred_element_type=jnp.float32)
    hid = dot(tl_ref[...], w1a[...]) + dot(th_ref[...], w1b[...]) + b1_ref[...]
    hid = jnp.maximum(hid, 0.0)
    out_ref[...] = dot(hid, w2_ref[...]) + b2_ref[...]


def _blk(minor):
    return pl.BlockSpec((BROW, minor), lambda i: (i, 0))


def _full(shape):
    return pl.BlockSpec(shape, lambda i: tuple(0 for _ in shape))


_PACKED = jax.ShapeDtypeStruct((NPQ, 128), jnp.float32)


def _embed_call(fpack, pbd, qbd, blo, bhi):
    return pl.pallas_call(
        _embed_body,
        grid=(GRID,),
        in_specs=[_blk(4 * IN_DIM), _full((4 * IN_DIM, 128)), _full((4 * IN_DIM, 128)),
                  _full((1, 128)), _full((1, 128))],
        out_specs=[_blk(128), _blk(128)],
        out_shape=[_PACKED, _PACKED],
    )(fpack, pbd, qbd, blo, bhi)


def _layer_call(residual, tl, th, nl, nh, dg, ws, blo, bhi):
    return pl.pallas_call(
        functools.partial(_layer_body, residual=residual),
        grid=(GRID,),
        in_specs=[_blk(128)] * 5 + [_full((128, 128))] * 8 + [_full((1, 128))] * 2,
        out_specs=[_blk(128), _blk(128)],
        out_shape=[_PACKED, _PACKED],
    )(tl, th, nl, nh, dg, *ws, blo, bhi)


def _head_call(tl, th, w1a, w1b, b1p, w2bd, b2p):
    return pl.pallas_call(
        _head_body,
        grid=(GRID,),
        in_specs=[_blk(128), _blk(128), _full((128, 128)), _full((128, 128)),
                  _full((1, 128)), _full((128, 4 * ODIM)), _full((1, 4 * ODIM))],
        out_specs=_blk(4 * ODIM),
        out_shape=jax.ShapeDtypeStruct((NPQ, 4 * ODIM), jnp.float32),
    )(tl, th, w1a, w1b, b1p, w2bd, b2p)


def kernel(features, edge_index, W_emb, b_emb, W_self, W_neigh, b_sage,
           bn_gamma, bn_beta, bn_mean, bn_var, W1, b1, W2, b2):
    ei = edge_index.astype(jnp.int32)
    pad = jnp.zeros((CH,), jnp.int32)
    srcp = jnp.concatenate([ei[0], pad])
    dstp = jnp.concatenate([ei[1], pad])

    scale = bn_gamma * lax.rsqrt(bn_var + 1e-5)
    bf = (b_sage - bn_mean) * scale + bn_beta
    Wsf = W_self * scale[:, :, None]
    Wnf = W_neigh * scale[:, :, None]
    eye4 = jnp.eye(4, dtype=jnp.float32)
    bd = lambda m: jnp.kron(eye4, m)
    layer_ws = []
    layer_bs = []
    for i in range(NLAYERS):
        ws = [bd(Wsf[i, :HALF, :HALF].T), bd(Wsf[i, :HALF, HALF:].T),
              bd(Wsf[i, HALF:, :HALF].T), bd(Wsf[i, HALF:, HALF:].T),
              bd(Wnf[i, :HALF, :HALF].T), bd(Wnf[i, :HALF, HALF:].T),
              bd(Wnf[i, HALF:, :HALF].T), bd(Wnf[i, HALF:, HALF:].T)]
        layer_ws.append(ws)
        layer_bs.append((jnp.tile(bf[i, :HALF], 4)[None, :],
                         jnp.tile(bf[i, HALF:], 4)[None, :]))
    pbd = bd(W_emb[:HALF, :].T)
    qbd = bd(W_emb[HALF:, :].T)
    eblo = jnp.tile(b_emb[:HALF], 4)[None, :]
    ebhi = jnp.tile(b_emb[HALF:], 4)[None, :]
    w1a = bd(W1[:, :HALF].T)
    w1b = bd(W1[:, HALF:].T)
    b1p = jnp.tile(b1, 4)[None, :]
    w2bd = bd(W2.T)
    b2p = jnp.tile(b2, 4)[None, :]

    fpack = jnp.pad(features, ((0, NP - N), (0, 0))).reshape(NPQ, 4 * IN_DIM)

    agg0k, aggk = _sc_kernels()
    hl, hh = _embed_call(fpack, pbd, qbd, eblo, ebhi)
    degrep = None
    for i in range(NLAYERS):
        if i == 0:
            nl, nh, deg = agg0k(hl.reshape(NP, HALF), hh.reshape(NP, HALF),
                                srcp, dstp)
            degrep = jnp.repeat(deg, HALF).reshape(NPQ, 128)
        else:
            nl, nh = aggk(hl.reshape(NP, HALF), hh.reshape(NP, HALF),
                          srcp, dstp)
        hl, hh = _layer_call(i > 0, hl, hh,
                             nl.reshape(NPQ, 128), nh.reshape(NPQ, 128),
                             degrep, layer_ws[i], *layer_bs[i])
    out = _head_call(hl, hh, w1a, w1b, b1p, w2bd, b2p)
    return out.reshape(NP, ODIM)[:N]


# speedup vs baseline: 13.3032x; 1.0555x over previous
import functools
import jax
import jax.numpy as jnp
from jax import lax
from jax.experimental import pallas as pl
from jax.experimental.pallas import tpu as pltpu
from jax.experimental.pallas import tpu_sc as plsc

N = 50000
E = 800000
IN_DIM = 12
HID = 64
HALF = 32
ODIM = 2
NLAYERS = 3

NP = 51200
NPQ = NP // 4

NC = 2
NS = 16
EPT = E // NS
CH = 400
NCHUNK = EPT // CH
RPT = 3128
LAST_BASE = N - RPT
NZ = RPT // CH
REM = RPT - NZ * CH
RPTP = NP // NS
NPAD_CH = (NP - N) // CH


def _agg_body(tab_lo, tab_hi, srcp, dstp, *refs, with_deg):
    if with_deg:
        (out_lo, out_hi, deg_out,
         src0, dst0, rows0, src1, dst1, rows1, ones_v, acc, acc_deg,
         semI0, semG0, semS0, semI1, semG1, semS1) = refs
    else:
        (out_lo, out_hi,
         src0, dst0, rows0, src1, dst1, rows1, acc,
         semI0, semG0, semS0, semI1, semG1, semS1) = refs
    c = lax.axis_index("c")
    s = lax.axis_index("s")
    zero16 = jnp.zeros((16,), jnp.float32)
    one16 = jnp.ones((16,), jnp.float32)

    def zrow(j, carry):
        rows0[j, pl.ds(0, 16)] = zero16
        rows0[j, pl.ds(16, 16)] = zero16
        return carry

    lax.fori_loop(0, CH, zrow, 0)

    base = pl.multiple_of(jnp.where(s == NS - 1, LAST_BASE, s * RPT), 8)

    def zcp(j, carry):
        pltpu.sync_copy(rows0, acc.at[pl.ds(base + j * CH, CH)])
        return carry

    lax.fori_loop(0, NZ, zcp, 0)
    pltpu.sync_copy(rows0.at[pl.ds(0, REM)], acc.at[pl.ds(base + NZ * CH, REM)])

    if with_deg:
        def fill0(j, carry):
            ones_v[pl.ds(j * 16, 16)] = zero16
            return carry

        lax.fori_loop(0, CH // 16, fill0, 0)
        basep = pl.multiple_of(s * RPTP, 8)

        def zdeg(j, carry):
            pltpu.sync_copy(ones_v, acc_deg.at[pl.ds(basep + j * CH, CH)])
            return carry

        lax.fori_loop(0, RPTP // CH, zdeg, 0)

        def fill1(j, carry):
            ones_v[pl.ds(j * 16, 16)] = one16
            return carry

        lax.fori_loop(0, CH // 16, fill1, 0)

    plsc.subcore_barrier()

    ebase = s * EPT
    slots = ((src0, dst0, rows0, semI0, semG0, semS0),
             (src1, dst1, rows1, semI1, semG1, semS1))

    def issue_i(k, sl):
        off = pl.multiple_of(ebase + k * CH, 8)
        pltpu.async_copy(srcp.at[pl.ds(off, CH)], sl[0], sl[3])
        pltpu.async_copy(dstp.at[pl.ds(off, CH)], sl[1], sl[3])

    def wait_i(sl):
        pltpu.make_async_copy(srcp.at[pl.ds(0, CH)], sl[0], sl[3]).wait()
        pltpu.make_async_copy(dstp.at[pl.ds(0, CH)], sl[1], sl[3]).wait()

    def issue_g(sl):
        @pl.when(c == 0)
        def _():
            pltpu.async_copy(tab_lo.at[sl[0]], sl[2], sl[4])

        @pl.when(c == 1)
        def _():
            pltpu.async_copy(tab_hi.at[sl[0]], sl[2], sl[4])

    def wait_g(sl):
        pltpu.make_async_copy(tab_lo.at[sl[0]], sl[2], sl[4]).wait()

    def issue_s(sl):
        pltpu.async_copy(sl[2], acc.at[sl[1]], sl[5], add=True)
        if with_deg:
            pltpu.async_copy(ones_v, acc_deg.at[sl[1]], sl[5], add=True)

    def wait_s(sl):
        pltpu.make_async_copy(sl[2], acc.at[sl[1]], sl[5]).wait()
        if with_deg:
            pltpu.make_async_copy(ones_v, acc_deg.at[sl[1]], sl[5]).wait()

    issue_i(0, slots[0])
    wait_i(slots[0])
    issue_g(slots[0])
    issue_i(1, slots[1])
    wait_g(slots[0])
    issue_s(slots[0])

    def pair(p, carry):
        for b in (1, 0):
            k = 2 * p + (1 if b == 1 else 2)
            sl = slots[b]
            ot = slots[1 - b]
            wait_i(sl)
            issue_g(sl)
            wait_s(ot)

            @pl.when(k + 1 < NCHUNK)
            def _():
                issue_i(k + 1, ot)

            wait_g(sl)
            issue_s(sl)
        return carry

    lax.fori_loop(0, (NCHUNK - 1) // 2, pair, 0)

    wait_s(slots[0])
    plsc.subcore_barrier()

    @pl.when(c == 0)
    def _():
        pltpu.sync_copy(acc.at[pl.ds(base, RPT)], out_lo.at[pl.ds(base, RPT)])

    @pl.when(c == 1)
    def _():
        pltpu.sync_copy(acc.at[pl.ds(base, RPT)], out_hi.at[pl.ds(base, RPT)])

    if with_deg:
        basep = pl.multiple_of(s * RPTP, 8)

        def wdeg(j, carry):
            pltpu.sync_copy(acc_deg.at[pl.ds(basep + j * CH, CH)], ones_v)
            pltpu.sync_copy(ones_v, deg_out.at[pl.ds(basep + j * CH, CH)])
            return carry

        lax.fori_loop(0, RPTP // CH, wdeg, 0)

    @pl.when(s == 0)
    def _():
        lax.fori_loop(0, CH, zrow, 0)

        def pz(j, carry):
            @pl.when(c == 0)
            def _():
                pltpu.sync_copy(rows0, out_lo.at[pl.ds(N + j * CH, CH)])

            @pl.when(c == 1)
            def _():
                pltpu.sync_copy(rows0, out_hi.at[pl.ds(N + j * CH, CH)])

            return carry

        lax.fori_loop(0, NPAD_CH, pz, 0)


@functools.cache
def _sc_kernels():
    mesh = plsc.VectorSubcoreMesh(core_axis_name="c", subcore_axis_name="s",
                                  num_cores=NC, num_subcores=NS)
    tab = jax.ShapeDtypeStruct((NP, HALF), jnp.float32)
    sems = [pltpu.SemaphoreType.DMA] * 6
    slot_bufs = [
        pltpu.VMEM((CH,), jnp.int32),
        pltpu.VMEM((CH,), jnp.int32),
        pltpu.VMEM((CH, HALF), jnp.float32),
        pltpu.VMEM((CH,), jnp.int32),
        pltpu.VMEM((CH,), jnp.int32),
        pltpu.VMEM((CH, HALF), jnp.float32),
    ]
    agg0 = functools.partial(
        pl.kernel,
        out_type=[tab, tab, jax.ShapeDtypeStruct((NP,), jnp.float32)],
        mesh=mesh,
        compiler_params=pltpu.CompilerParams(use_tc_tiling_on_sc=False),
        scratch_types=slot_bufs + [
            pltpu.VMEM((CH,), jnp.float32),
            pltpu.VMEM_SHARED((N, HALF), jnp.float32),
            pltpu.VMEM_SHARED((NP,), jnp.float32),
        ] + sems,
    )(functools.partial(_agg_body, with_deg=True))
    agg = functools.partial(
        pl.kernel,
        out_type=[tab, tab],
        mesh=mesh,
        compiler_params=pltpu.CompilerParams(use_tc_tiling_on_sc=False),
        scratch_types=slot_bufs + [
            pltpu.VMEM_SHARED((N, HALF), jnp.float32),
        ] + sems,
    )(functools.partial(_agg_body, with_deg=False))
    return agg0, agg


BROW = 320
GRID = NPQ // BROW


def _embed_body(f_ref, p_ref, q_ref, blo_ref, bhi_ref, lo_ref, hi_ref):
    f = f_ref[...]
    lo_ref[...] = jnp.dot(f, p_ref[...], preferred_element_type=jnp.float32) + blo_ref[...]
    hi_ref[...] = jnp.dot(f, q_ref[...], preferred_element_type=jnp.float32) + bhi_ref[...]


def _layer_body(tl_ref, th_ref, nl_ref, nh_ref, dg_ref,
                sa, sb, sc_, sd, na, nb, ncc, nd, blo_ref, bhi_ref,
                lo_ref, hi_ref, *, residual):
    tl = tl_ref[...]
    th = th_ref[...]
    nl = nl_ref[...]
    nh = nh_ref[...]
    invd = 1.0 / jnp.maximum(dg_ref[...], 1.0)
    dot = functools.partial(jnp.dot, preferred_element_type=jnp.float32)
    xlo = dot(tl, sa[...]) + dot(th, sb[...]) + (dot(nl, na[...]) + dot(nh, nb[...])) * invd + blo_ref[...]
    xhi = dot(tl, sc_[...]) + dot(th, sd[...]) + (dot(nl, ncc[...]) + dot(nh, nd[...])) * invd + bhi_ref[...]
    xlo = jnp.maximum(xlo, 0.0)
    xhi = jnp.maximum(xhi, 0.0)
    if residual:
        xlo = xlo + tl
        xhi = xhi + th
    lo_ref[...] = xlo
    hi_ref[...] = xhi


def _layer_head_body(tl_ref, th_ref, nl_ref, nh_ref, dg_ref,
                     sa, sb, sc_, sd, na, nb, ncc, nd, blo_ref, bhi_ref,
                     w1a, w1b, b1_ref, w2_ref, b2_ref, out_ref):
    tl = tl_ref[...]
    th = th_ref[...]
    nl = nl_ref[...]
    nh = nh_ref[...]
    invd = 1.0 / jnp.maximum(dg_ref[...], 1.0)
    dot = functools.partial(jnp.dot, preferred_element_type=jnp.float32)
    xlo = dot(tl, sa[...]) + dot(th, sb[...]) + (dot(nl, na[...]) + dot(nh, nb[...])) * invd + blo_ref[...]
    xhi = dot(tl, sc_[...]) + dot(th, sd[...]) + (dot(nl, ncc[...]) + dot(nh, nd[...])) * invd + bhi_ref[...]
    xlo = jnp.maximum(xlo, 0.0) + tl
    xhi = jnp.maximum(xhi, 0.0) + th
    hid = dot(xlo, w1a[...]) + dot(xhi, w1b[...]) + b1_ref[...]
    hid = jnp.maximum(hid, 0.0)
    out_ref[...] = dot(hid, w2_ref[...]) + b2_ref[...]


def _blk(minor):
    return pl.BlockSpec((BROW, minor), lambda i: (i, 0))


def _full(shape):
    return pl.BlockSpec(shape, lambda i: tuple(0 for _ in shape))


_PACKED = jax.ShapeDtypeStruct((NPQ, 128), jnp.float32)


def _embed_call(fpack, pbd, qbd, blo, bhi):
    return pl.pallas_call(
        _embed_body,
        grid=(GRID,),
        in_specs=[_blk(4 * IN_DIM), _full((4 * IN_DIM, 128)), _full((4 * IN_DIM, 128)),
                  _full((1, 128)), _full((1, 128))],
        out_specs=[_blk(128), _blk(128)],
        out_shape=[_PACKED, _PACKED],
    )(fpack, pbd, qbd, blo, bhi)


def _layer_call(residual, tl, th, nl, nh, dg, ws, blo, bhi):
    return pl.pallas_call(
        functools.partial(_layer_body, residual=residual),
        grid=(GRID,),
        in_specs=[_blk(128)] * 5 + [_full((128, 128))] * 8 + [_full((1, 128))] * 2,
        out_specs=[_blk(128), _blk(128)],
        out_shape=[_PACKED, _PACKED],
    )(tl, th, nl, nh, dg, *ws, blo, bhi)


def _layer_head_call(tl, th, nl, nh, dg, ws, blo, bhi, w1a, w1b, b1p, w2bd, b2p):
    return pl.pallas_call(
        _layer_head_body,
        grid=(GRID,),
        in_specs=[_blk(128)] * 5 + [_full((128, 128))] * 8 + [_full((1, 128))] * 2
                 + [_full((128, 128)), _full((128, 128)), _full((1, 128)),
                    _full((128, 4 * ODIM)), _full((1, 4 * ODIM))],
        out_specs=_blk(4 * ODIM),
        out_shape=jax.ShapeDtypeStruct((NPQ, 4 * ODIM), jnp.float32),
    )(tl, th, nl, nh, dg, *ws, blo, bhi, w1a, w1b, b1p, w2bd, b2p)


def kernel(features, edge_index, W_emb, b_emb, W_self, W_neigh, b_sage,
           bn_gamma, bn_beta, bn_mean, bn_var, W1, b1, W2, b2):
    ei = edge_index.astype(jnp.int32)
    srcp = ei[0]
    dstp = ei[1]

    scale = bn_gamma * lax.rsqrt(bn_var + 1e-5)
    bf = (b_sage - bn_mean) * scale + bn_beta
    Wsf = W_self * scale[:, :, None]
    Wnf = W_neigh * scale[:, :, None]
    eye4 = jnp.eye(4, dtype=jnp.float32)
    bd = lambda m: jnp.kron(eye4, m)
    layer_ws = []
    layer_bs = []
    for i in range(NLAYERS):
        ws = [bd(Wsf[i, :HALF, :HALF].T), bd(Wsf[i, :HALF, HALF:].T),
              bd(Wsf[i, HALF:, :HALF].T), bd(Wsf[i, HALF:, HALF:].T),
              bd(Wnf[i, :HALF, :HALF].T), bd(Wnf[i, :HALF, HALF:].T),
              bd(Wnf[i, HALF:, :HALF].T), bd(Wnf[i, HALF:, HALF:].T)]
        layer_ws.append(ws)
        layer_bs.append((jnp.tile(bf[i, :HALF], 4)[None, :],
                         jnp.tile(bf[i, HALF:], 4)[None, :]))
    pbd = bd(W_emb[:HALF, :].T)
    qbd = bd(W_emb[HALF:, :].T)
    eblo = jnp.tile(b_emb[:HALF], 4)[None, :]
    ebhi = jnp.tile(b_emb[HALF:], 4)[None, :]
    w1a = bd(W1[:, :HALF].T)
    w1b = bd(W1[:, HALF:].T)
    b1p = jnp.tile(b1, 4)[None, :]
    w2bd = bd(W2.T)
    b2p = jnp.tile(b2, 4)[None, :]

    fpack = jnp.pad(features.reshape(N // 4, 4 * IN_DIM), ((0, NPQ - N // 4), (0, 0)))

    agg0k, aggk = _sc_kernels()
    hl, hh = _embed_call(fpack, pbd, qbd, eblo, ebhi)
    degrep = None
    for i in range(NLAYERS):
        if i == 0:
            nl, nh, deg = agg0k(hl.reshape(NP, HALF), hh.reshape(NP, HALF),
                                srcp, dstp)
            degrep = jnp.repeat(deg, HALF).reshape(NPQ, 128)
        else:
            nl, nh = aggk(hl.reshape(NP, HALF), hh.reshape(NP, HALF),
                          srcp, dstp)
        nlp = nl.reshape(NPQ, 128)
        nhp = nh.reshape(NPQ, 128)
        if i < NLAYERS - 1:
            hl, hh = _layer_call(i > 0, hl, hh, nlp, nhp,
                                 degrep, layer_ws[i], *layer_bs[i])
        else:
            out = _layer_head_call(hl, hh, nlp, nhp, degrep,
                                   layer_ws[i], *layer_bs[i],
                                   w1a=w1a, w1b=w1b, b1p=b1p, w2bd=w2bd, b2p=b2p)
    return out.reshape(NP, ODIM)[:N]


# speedup vs baseline: 14.0730x; 1.0579x over previous
import functools
import jax
import jax.numpy as jnp
from jax import lax
from jax.experimental import pallas as pl
from jax.experimental.pallas import tpu as pltpu
from jax.experimental.pallas import tpu_sc as plsc

N = 50000
E = 800000
IN_DIM = 12
HID = 64
HALF = 32
ODIM = 2
NLAYERS = 3

NP = 51200
NPQ = NP // 4

NC = 2
NS = 16
EPT = E // NS
CH = 400
NCHUNK = EPT // CH
RPT = 3128
LAST_BASE = N - RPT
NZ = RPT // CH
REM = RPT - NZ * CH
RPTP = NP // NS
NPAD_CH = (NP - N) // CH


def _agg_body(tab_lo, tab_hi, srcp, dstp, *refs, with_deg):
    if with_deg:
        (out_lo, out_hi, deg_out,
         src0, dst0, rows0, src1, dst1, rows1, ones_v, acc, acc_deg,
         semI0, semG0, semS0, semI1, semG1, semS1) = refs
    else:
        (out_lo, out_hi,
         src0, dst0, rows0, src1, dst1, rows1, acc,
         semI0, semG0, semS0, semI1, semG1, semS1) = refs
    c = lax.axis_index("c")
    s = lax.axis_index("s")
    zero16 = jnp.zeros((16,), jnp.float32)
    one16 = jnp.ones((16,), jnp.float32)

    def zrow(j, carry):
        rows0[j, pl.ds(0, 16)] = zero16
        rows0[j, pl.ds(16, 16)] = zero16
        return carry

    lax.fori_loop(0, CH, zrow, 0)

    base = pl.multiple_of(jnp.where(s == NS - 1, LAST_BASE, s * RPT), 8)

    def zcp(j, carry):
        pltpu.sync_copy(rows0, acc.at[pl.ds(base + j * CH, CH)])
        return carry

    lax.fori_loop(0, NZ, zcp, 0)
    pltpu.sync_copy(rows0.at[pl.ds(0, REM)], acc.at[pl.ds(base + NZ * CH, REM)])

    if with_deg:
        def fill0(j, carry):
            ones_v[pl.ds(j * 16, 16)] = zero16
            return carry

        lax.fori_loop(0, CH // 16, fill0, 0)
        basep = pl.multiple_of(s * RPTP, 8)

        def zdeg(j, carry):
            pltpu.sync_copy(ones_v, acc_deg.at[pl.ds(basep + j * CH, CH)])
            return carry

        lax.fori_loop(0, RPTP // CH, zdeg, 0)

        def fill1(j, carry):
            ones_v[pl.ds(j * 16, 16)] = one16
            return carry

        lax.fori_loop(0, CH // 16, fill1, 0)

    plsc.subcore_barrier()

    ebase = s * EPT
    slots = ((src0, dst0, rows0, semI0, semG0, semS0),
             (src1, dst1, rows1, semI1, semG1, semS1))

    def issue_i(k, sl):
        off = pl.multiple_of(ebase + k * CH, 8)
        pltpu.async_copy(srcp.at[pl.ds(off, CH)], sl[0], sl[3])
        pltpu.async_copy(dstp.at[pl.ds(off, CH)], sl[1], sl[3])

    def wait_i(sl):
        pltpu.make_async_copy(srcp.at[pl.ds(0, CH)], sl[0], sl[3]).wait()
        pltpu.make_async_copy(dstp.at[pl.ds(0, CH)], sl[1], sl[3]).wait()

    def issue_g(sl):
        @pl.when(c == 0)
        def _():
            pltpu.async_copy(tab_lo.at[sl[0]], sl[2], sl[4])

        @pl.when(c == 1)
        def _():
            pltpu.async_copy(tab_hi.at[sl[0]], sl[2], sl[4])

    def wait_g(sl):
        pltpu.make_async_copy(tab_lo.at[sl[0]], sl[2], sl[4]).wait()

    def issue_s(sl):
        pltpu.async_copy(sl[2], acc.at[sl[1]], sl[5], add=True)
        if with_deg:
            pltpu.async_copy(ones_v, acc_deg.at[sl[1]], sl[5], add=True)

    def wait_s(sl):
        pltpu.make_async_copy(sl[2], acc.at[sl[1]], sl[5]).wait()
        if with_deg:
            pltpu.make_async_copy(ones_v, acc_deg.at[sl[1]], sl[5]).wait()

    issue_i(0, slots[0])
    wait_i(slots[0])
    issue_g(slots[0])
    issue_i(1, slots[1])
    wait_g(slots[0])
    issue_s(slots[0])

    def pair(p, carry):
        for b in (1, 0):
            k = 2 * p + (1 if b == 1 else 2)
            sl = slots[b]
            ot = slots[1 - b]
            wait_i(sl)
            issue_g(sl)
            wait_s(ot)

            @pl.when(k + 1 < NCHUNK)
            def _():
                issue_i(k + 1, ot)

            wait_g(sl)
            issue_s(sl)
        return carry

    lax.fori_loop(0, (NCHUNK - 1) // 2, pair, 0)

    wait_s(slots[0])
    plsc.subcore_barrier()

    @pl.when(c == 0)
    def _():
        pltpu.sync_copy(acc.at[pl.ds(base, RPT)], out_lo.at[pl.ds(base, RPT)])

    @pl.when(c == 1)
    def _():
        pltpu.sync_copy(acc.at[pl.ds(base, RPT)], out_hi.at[pl.ds(base, RPT)])

    if with_deg:
        basep = pl.multiple_of(s * RPTP, 8)

        def wdeg(j, carry):
            pltpu.sync_copy(acc_deg.at[pl.ds(basep + j * CH, CH)], ones_v)
            pltpu.sync_copy(ones_v, deg_out.at[pl.ds(basep + j * CH, CH)])
            return carry

        lax.fori_loop(0, RPTP // CH, wdeg, 0)

    @pl.when(s == 0)
    def _():
        lax.fori_loop(0, CH, zrow, 0)

        def pz(j, carry):
            @pl.when(c == 0)
            def _():
                pltpu.sync_copy(rows0, out_lo.at[pl.ds(N + j * CH, CH)])

            @pl.when(c == 1)
            def _():
                pltpu.sync_copy(rows0, out_hi.at[pl.ds(N + j * CH, CH)])

            return carry

        lax.fori_loop(0, NPAD_CH, pz, 0)


@functools.cache
def _sc_kernels():
    mesh = plsc.VectorSubcoreMesh(core_axis_name="c", subcore_axis_name="s",
                                  num_cores=NC, num_subcores=NS)
    tab = jax.ShapeDtypeStruct((NP, HALF), jnp.float32)
    sems = [pltpu.SemaphoreType.DMA] * 6
    slot_bufs = [
        pltpu.VMEM((CH,), jnp.int32),
        pltpu.VMEM((CH,), jnp.int32),
        pltpu.VMEM((CH, HALF), jnp.float32),
        pltpu.VMEM((CH,), jnp.int32),
        pltpu.VMEM((CH,), jnp.int32),
        pltpu.VMEM((CH, HALF), jnp.float32),
    ]
    agg0 = functools.partial(
        pl.kernel,
        out_type=[tab, tab, jax.ShapeDtypeStruct((NP,), jnp.float32)],
        mesh=mesh,
        compiler_params=pltpu.CompilerParams(use_tc_tiling_on_sc=False),
        scratch_types=slot_bufs + [
            pltpu.VMEM((CH,), jnp.float32),
            pltpu.VMEM_SHARED((N, HALF), jnp.float32),
            pltpu.VMEM_SHARED((NP,), jnp.float32),
        ] + sems,
    )(functools.partial(_agg_body, with_deg=True))
    agg = functools.partial(
        pl.kernel,
        out_type=[tab, tab],
        mesh=mesh,
        compiler_params=pltpu.CompilerParams(use_tc_tiling_on_sc=False),
        scratch_types=slot_bufs + [
            pltpu.VMEM_SHARED((N, HALF), jnp.float32),
        ] + sems,
    )(functools.partial(_agg_body, with_deg=False))
    return agg0, agg


BROW = 640
GRID = NPQ // BROW


def _embed_body(f_ref, p_ref, q_ref, blo_ref, bhi_ref, lo_ref, hi_ref):
    f = f_ref[...]
    lo_ref[...] = jnp.dot(f, p_ref[...], preferred_element_type=jnp.float32) + blo_ref[...]
    hi_ref[...] = jnp.dot(f, q_ref[...], preferred_element_type=jnp.float32) + bhi_ref[...]


def _layer_body(tl_ref, th_ref, nl_ref, nh_ref, dg_ref,
                sa, sb, sc_, sd, na, nb, ncc, nd, blo_ref, bhi_ref,
                lo_ref, hi_ref, *, residual):
    tl = tl_ref[...]
    th = th_ref[...]
    nl = nl_ref[...]
    nh = nh_ref[...]
    invd = 1.0 / jnp.maximum(dg_ref[...], 1.0)
    dot = functools.partial(jnp.dot, preferred_element_type=jnp.float32)
    xlo = dot(tl, sa[...]) + dot(th, sb[...]) + (dot(nl, na[...]) + dot(nh, nb[...])) * invd + blo_ref[...]
    xhi = dot(tl, sc_[...]) + dot(th, sd[...]) + (dot(nl, ncc[...]) + dot(nh, nd[...])) * invd + bhi_ref[...]
    xlo = jnp.maximum(xlo, 0.0)
    xhi = jnp.maximum(xhi, 0.0)
    if residual:
        xlo = xlo + tl
        xhi = xhi + th
    lo_ref[...] = xlo
    hi_ref[...] = xhi


def _layer_head_body(tl_ref, th_ref, nl_ref, nh_ref, dg_ref,
                     sa, sb, sc_, sd, na, nb, ncc, nd, blo_ref, bhi_ref,
                     w1a, w1b, b1_ref, w2_ref, b2_ref, out_ref):
    tl = tl_ref[...]
    th = th_ref[...]
    nl = nl_ref[...]
    nh = nh_ref[...]
    invd = 1.0 / jnp.maximum(dg_ref[...], 1.0)
    dot = functools.partial(jnp.dot, preferred_element_type=jnp.float32)
    xlo = dot(tl, sa[...]) + dot(th, sb[...]) + (dot(nl, na[...]) + dot(nh, nb[...])) * invd + blo_ref[...]
    xhi = dot(tl, sc_[...]) + dot(th, sd[...]) + (dot(nl, ncc[...]) + dot(nh, nd[...])) * invd + bhi_ref[...]
    xlo = jnp.maximum(xlo, 0.0) + tl
    xhi = jnp.maximum(xhi, 0.0) + th
    hid = dot(xlo, w1a[...]) + dot(xhi, w1b[...]) + b1_ref[...]
    hid = jnp.maximum(hid, 0.0)
    out_ref[...] = dot(hid, w2_ref[...]) + b2_ref[...]


def _blk(minor):
    return pl.BlockSpec((BROW, minor), lambda i: (i, 0))


def _full(shape):
    return pl.BlockSpec(shape, lambda i: tuple(0 for _ in shape))


_PACKED = jax.ShapeDtypeStruct((NPQ, 128), jnp.float32)


def _embed_call(fpack, pbd, qbd, blo, bhi):
    return pl.pallas_call(
        _embed_body,
        grid=(GRID,),
        in_specs=[_blk(4 * IN_DIM), _full((4 * IN_DIM, 128)), _full((4 * IN_DIM, 128)),
                  _full((1, 128)), _full((1, 128))],
        out_specs=[_blk(128), _blk(128)],
        out_shape=[_PACKED, _PACKED],
    )(fpack, pbd, qbd, blo, bhi)


def _layer_call(residual, tl, th, nl, nh, dg, ws, blo, bhi):
    return pl.pallas_call(
        functools.partial(_layer_body, residual=residual),
        grid=(GRID,),
        in_specs=[_blk(128)] * 5 + [_full((128, 128))] * 8 + [_full((1, 128))] * 2,
        out_specs=[_blk(128), _blk(128)],
        out_shape=[_PACKED, _PACKED],
    )(tl, th, nl, nh, dg, *ws, blo, bhi)


def _layer_head_call(tl, th, nl, nh, dg, ws, blo, bhi, w1a, w1b, b1p, w2bd, b2p):
    return pl.pallas_call(
        _layer_head_body,
        grid=(GRID,),
        in_specs=[_blk(128)] * 5 + [_full((128, 128))] * 8 + [_full((1, 128))] * 2
                 + [_full((128, 128)), _full((128, 128)), _full((1, 128)),
                    _full((128, 4 * ODIM)), _full((1, 4 * ODIM))],
        out_specs=_blk(4 * ODIM),
        out_shape=jax.ShapeDtypeStruct((NPQ, 4 * ODIM), jnp.float32),
    )(tl, th, nl, nh, dg, *ws, blo, bhi, w1a, w1b, b1p, w2bd, b2p)


def kernel(features, edge_index, W_emb, b_emb, W_self, W_neigh, b_sage,
           bn_gamma, bn_beta, bn_mean, bn_var, W1, b1, W2, b2):
    ei = edge_index.astype(jnp.int32)
    srcp = ei[0]
    dstp = ei[1]

    scale = bn_gamma * lax.rsqrt(bn_var + 1e-5)
    bf = (b_sage - bn_mean) * scale + bn_beta
    Wsf = W_self * scale[:, :, None]
    Wnf = W_neigh * scale[:, :, None]
    eye4 = jnp.eye(4, dtype=jnp.float32)
    bd = lambda m: jnp.kron(eye4, m)
    layer_ws = []
    layer_bs = []
    for i in range(NLAYERS):
        ws = [bd(Wsf[i, :HALF, :HALF].T), bd(Wsf[i, :HALF, HALF:].T),
              bd(Wsf[i, HALF:, :HALF].T), bd(Wsf[i, HALF:, HALF:].T),
              bd(Wnf[i, :HALF, :HALF].T), bd(Wnf[i, :HALF, HALF:].T),
              bd(Wnf[i, HALF:, :HALF].T), bd(Wnf[i, HALF:, HALF:].T)]
        layer_ws.append(ws)
        layer_bs.append((jnp.tile(bf[i, :HALF], 4)[None, :],
                         jnp.tile(bf[i, HALF:], 4)[None, :]))
    pbd = bd(W_emb[:HALF, :].T)
    qbd = bd(W_emb[HALF:, :].T)
    eblo = jnp.tile(b_emb[:HALF], 4)[None, :]
    ebhi = jnp.tile(b_emb[HALF:], 4)[None, :]
    w1a = bd(W1[:, :HALF].T)
    w1b = bd(W1[:, HALF:].T)
    b1p = jnp.tile(b1, 4)[None, :]
    w2bd = bd(W2.T)
    b2p = jnp.tile(b2, 4)[None, :]

    fpack = jnp.pad(features.reshape(N // 4, 4 * IN_DIM), ((0, NPQ - N // 4), (0, 0)))

    agg0k, aggk = _sc_kernels()
    hl, hh = _embed_call(fpack, pbd, qbd, eblo, ebhi)
    degrep = None
    for i in range(NLAYERS):
        if i == 0:
            nl, nh, deg = agg0k(hl.reshape(NP, HALF), hh.reshape(NP, HALF),
                                srcp, dstp)
            degrep = jnp.repeat(deg, HALF).reshape(NPQ, 128)
        else:
            nl, nh = aggk(hl.reshape(NP, HALF), hh.reshape(NP, HALF),
                          srcp, dstp)
        nlp = nl.reshape(NPQ, 128)
        nhp = nh.reshape(NPQ, 128)
        if i < NLAYERS - 1:
            hl, hh = _layer_call(i > 0, hl, hh, nlp, nhp,
                                 degrep, layer_ws[i], *layer_bs[i])
        else:
            out = _layer_head_call(hl, hh, nlp, nhp, degrep,
                                   layer_ws[i], *layer_bs[i],
                                   w1a=w1a, w1b=w1b, b1p=b1p, w2bd=w2bd, b2p=b2p)
    return out.reshape(NP, ODIM)[:N]
